# baseline (speedup 1.0000x reference)
import functools

import jax
import jax.numpy as jnp
from jax import lax
from jax.experimental import pallas as pl
from jax.experimental.pallas import tpu as pltpu
from jax.experimental.pallas import tpu_sc as plsc

_G = 8


@functools.lru_cache(maxsize=None)
def _make_group_gather(V, D, B):
    info = plsc.get_sparse_core_info()
    NC, NS = info.num_cores, info.num_subcores
    NW = NC * NS
    assert B % NW == 0
    b_per_w = B // NW
    mesh = plsc.VectorSubcoreMesh(core_axis_name="c", subcore_axis_name="s")
    RING = 16

    @functools.partial(
        pl.kernel,
        mesh=mesh,
        out_type=jax.ShapeDtypeStruct((B * _G, D), jnp.float32),
        scratch_types=[
            pltpu.VMEM((b_per_w + 16,), jnp.int32),
            pltpu.VMEM((RING * _G, D), jnp.float32),
            pltpu.SemaphoreType.DMA,
            pltpu.SemaphoreType.DMA,
        ],
    )
    def k(table, idx_hbm, out_hbm, idx_v, ring, gsem, osem):
        wid = lax.axis_index("s") * NC + lax.axis_index("c")
        base = wid * b_per_w
        pltpu.sync_copy(
            idx_hbm.at[pl.ds(base, b_per_w)], idx_v.at[pl.ds(0, b_per_w)]
        )

        def fetch(t):
            n = idx_v[pl.ds(t, 16)][0]
            g = pl.multiple_of((n // _G) * _G, _G)
            s = pl.multiple_of(lax.rem(t, RING) * _G, _G)
            return pltpu.make_async_copy(
                table.at[pl.ds(g, _G), :], ring.at[pl.ds(s, _G), :], gsem
            )

        def flush(t):
            s = pl.multiple_of(lax.rem(t, RING) * _G, _G)
            pos = pl.multiple_of((base + t) * _G, _G)
            return pltpu.make_async_copy(
                ring.at[pl.ds(s, _G), :], out_hbm.at[pl.ds(pos, _G), :], osem
            )

        for t in range(RING):
            fetch(t).start()

        def body(t, carry):
            fetch(t).wait()
            flush(t).start()
            flush(t).wait()

            @pl.when(t + RING < b_per_w)
            def _():
                fetch(t + RING).start()

            return carry

        lax.fori_loop(0, b_per_w, body, 0)

    return k


def kernel(memory, nodes):
    nodes = nodes.astype(jnp.int32)
    V, D = memory.shape
    B = nodes.shape[0]
    flat = _make_group_gather(V, D, B)(memory, nodes)
    groups = flat.reshape(B, _G, D)
    k = nodes % _G
    out = groups[:, 0, :]
    for j in range(1, _G):
        out = jnp.where((k == j)[:, None], groups[:, j, :], out)
    return out


# speedup vs baseline: 1.3434x; 1.3434x over previous
import functools

import jax
import jax.numpy as jnp
from jax import lax
from jax.experimental import pallas as pl
from jax.experimental.pallas import tpu as pltpu
from jax.experimental.pallas import tpu_sc as plsc


@functools.lru_cache(maxsize=None)
def _make_row_gather(V, W, B):
    info = plsc.get_sparse_core_info()
    NC, NS = info.num_cores, info.num_subcores
    NW = NC * NS
    assert B % NW == 0
    b_per_w = B // NW
    mesh = plsc.VectorSubcoreMesh(core_axis_name="c", subcore_axis_name="s")

    @functools.partial(
        pl.kernel,
        mesh=mesh,
        out_type=jax.ShapeDtypeStruct((B, W), jnp.float32),
        scratch_types=[
            pltpu.VMEM((b_per_w,), jnp.int32),
            pltpu.VMEM((b_per_w, W), jnp.float32),
            pltpu.SemaphoreType.DMA,
        ],
    )
    def k(table, idx_hbm, out_hbm, idx_v, rows_v, sem):
        wid = lax.axis_index("s") * NC + lax.axis_index("c")
        base = wid * b_per_w
        pltpu.sync_copy(idx_hbm.at[pl.ds(base, b_per_w)], idx_v)
        pltpu.async_copy(table.at[idx_v], rows_v, sem).wait()
        pltpu.sync_copy(rows_v, out_hbm.at[pl.ds(base, b_per_w), :])

    return k


def kernel(memory, nodes):
    nodes = nodes.astype(jnp.int32)
    V, D = memory.shape
    mem128 = jnp.concatenate([memory, jnp.zeros_like(memory)], axis=1)
    rows = _make_row_gather(V, 2 * D, nodes.shape[0])(mem128, nodes)
    return rows[:, :D]
